# baseline (speedup 1.0000x reference)
import functools

import jax
import jax.numpy as jnp
from jax import lax
from jax.experimental import pallas as pl
from jax.experimental.pallas import tpu as pltpu
from jax.experimental.pallas import tpu_sc as plsc

N = 10000
E = 106666
D = 128
R = 3

NC = 2
NS = 16
NW = NC * NS

CH = 128
NCH = 27
EPT = NCH * CH
EPAD = EPT * NW

NPAD = 10240
RPT = NPAD // NS
NBUF = 2
DS = 0
ZR = 16

BN = 2048
GRID = NPAD // BN


def _make_sc_edge(compute_cnt: bool):
  mesh = plsc.VectorSubcoreMesh(core_axis_name="c", subcore_axis_name="s",
                                num_cores=NC, num_subcores=NS)

  out_type = [jax.ShapeDtypeStruct((NC, R, NPAD, D), jnp.float32)]
  if compute_cnt:
    out_type.append(jax.ShapeDtypeStruct((NC * R * NPAD,), jnp.float32))

  scratch = (
      [pltpu.VMEM((NCH, CH), jnp.int32)] * 2 +
      [pltpu.VMEM((CH, D), jnp.float32)] * NBUF +
      [pltpu.VMEM((ZR, D), jnp.float32)] +
      [pltpu.VMEM_SHARED((NPAD, D), jnp.float32)] +
      [pltpu.SemaphoreType.DMA] * (2 * NBUF + 2)
  )
  if compute_cnt:
    scratch += [
        pltpu.VMEM((CH,), jnp.float32),
        pltpu.VMEM((RPT,), jnp.float32),
        pltpu.SemaphoreType.DMA,
        pltpu.VMEM_SHARED((NPAD,), jnp.float32),
        pltpu.VMEM_SHARED((NPAD,), jnp.float32),
        pltpu.VMEM_SHARED((NPAD,), jnp.float32),
    ]

  def body(*refs):
    (htab, s0, s1, s2, d0, d1, d2) = refs[:7]
    pos = 7
    agg_out = refs[pos]; pos += 1
    if compute_cnt:
      cnt_out = refs[pos]; pos += 1
    src_v, dst_v = refs[pos:pos + 2]
    pos += 2
    rows = refs[pos:pos + NBUF]
    pos += NBUF
    zeros_v, agg_sh = refs[pos:pos + 2]
    pos += 2
    gsem = refs[pos:pos + NBUF]
    pos += NBUF
    ssem = refs[pos:pos + NBUF]
    pos += NBUF
    zsem, xsem = refs[pos:pos + 2]
    pos += 2
    if compute_cnt:
      ones_v, cstage_v, csem, c0_sh, c1_sh, c2_sh = refs[pos:pos + 6]
      cnt_shs = (c0_sh, c1_sh, c2_sh)

    c = lax.axis_index("c")
    tid = lax.axis_index("s")
    wid = tid * NC + c

    zv = jnp.zeros((16,), jnp.float32)

    def zrow(i, carry):
      for j in range(D // 16):
        zeros_v[i, pl.ds(j * 16, 16)] = zv
      return carry

    lax.fori_loop(0, ZR, zrow, 0)

    if compute_cnt:
      ov = jnp.full((16,), 1.0, jnp.float32)
      for j in range(CH // 16):
        ones_v[pl.ds(j * 16, 16)] = ov

    ss = (s0, s1, s2)
    ds = (d0, d1, d2)

    pltpu.sync_copy(ss[0].at[wid], src_v)
    pltpu.sync_copy(ds[0].at[wid], dst_v)

    for r in range(R):
      if r == 0:
        zd = [
            pltpu.async_copy(
                zeros_v, agg_sh.at[pl.ds(tid * RPT + k * ZR, ZR)], zsem)
            for k in range(RPT // ZR)
        ]
        for d in zd:
          d.wait()
      if compute_cnt:
        czd = [
            pltpu.async_copy(
                zeros_v.at[k % ZR],
                cnt_shs[r].at[pl.ds(tid * RPT + k * D, D)], zsem)
            for k in range(RPT // D)
        ]
        for d in czd:
          d.wait()

      gd = [None] * NCH
      for j in range(min(NBUF - DS, NCH)):
        gd[j] = pltpu.async_copy(
            htab.at[src_v.at[j]], rows[j % NBUF], gsem[j % NBUF])
      plsc.subcore_barrier()

      cds = []
      for j in range(NCH):
        b = j % NBUF
        gd[j].wait()
        if compute_cnt:
          cds.append(pltpu.async_copy(
              ones_v, cnt_shs[r].at[dst_v.at[j]], csem, add=True))
        pltpu.async_copy(
            rows[b], agg_sh.at[dst_v.at[j]], ssem[b], add=True).wait()
        if j + NBUF < NCH:
          gd[j + NBUF] = pltpu.async_copy(
              htab.at[src_v.at[j + NBUF]], rows[b], gsem[b])
      for d in cds:
        d.wait()
      plsc.subcore_barrier()

      ipf = []
      if r + 1 < R:
        ipf.append(pltpu.async_copy(ss[r + 1].at[wid], src_v, xsem))
        ipf.append(pltpu.async_copy(ds[r + 1].at[wid], dst_v, xsem))
      nd = RPT // CH
      dd = [None] * nd
      od = [None] * nd
      zd2 = []
      dd[0] = pltpu.async_copy(
          agg_sh.at[pl.ds(tid * RPT, CH)], rows[0], gsem[0])
      for k in range(nd):
        b = k % NBUF
        dd[k].wait()
        if r + 1 < R:
          for z in range(CH // ZR):
            zd2.append(pltpu.async_copy(
                zeros_v,
                agg_sh.at[pl.ds(tid * RPT + k * CH + z * ZR, ZR)], zsem))
        od[k] = pltpu.async_copy(
            rows[b], agg_out.at[c, r, pl.ds(tid * RPT + k * CH, CH)], ssem[b])
        if k + 1 < nd:
          if k - 1 >= 0:
            od[k - 1].wait()
          nb = (k + 1) % NBUF
          dd[k + 1] = pltpu.async_copy(
              agg_sh.at[pl.ds(tid * RPT + (k + 1) * CH, CH)], rows[nb],
              gsem[nb])
      for k in range(max(0, nd - 2), nd):
        od[k].wait()
      for d in zd2:
        d.wait()
      for d in ipf:
        d.wait()
      if compute_cnt:
        pltpu.sync_copy(cnt_shs[r].at[pl.ds(tid * RPT, RPT)], cstage_v)
        base = (c * R + r) * NPAD + tid * RPT
        pltpu.sync_copy(cstage_v, cnt_out.at[pl.ds(base, RPT)])

  out = tuple(out_type) if compute_cnt else out_type[0]
  return pl.kernel(body, out_type=out, mesh=mesh, scratch_types=scratch)


_sc_edge_cnt = _make_sc_edge(True)
_sc_edge = _make_sc_edge(False)


def _full(shape):
  return pl.BlockSpec(shape, lambda i: tuple(0 for _ in shape))


def _tc_layer_body(h_ref, wroot_ref, b_ref, cnt_ref, agg_ref, wrel_ref,
                   h_out):
  h = h_ref[...]
  acc = jnp.dot(h, wroot_ref[...], preferred_element_type=jnp.float32)
  acc = acc + b_ref[...]
  for r in range(R):
    a = agg_ref[0, r] + agg_ref[1, r]
    cc = cnt_ref[0, r] + cnt_ref[1, r]
    inv = 1.0 / jnp.maximum(cc, 1.0)
    m = a * inv
    acc = acc + jnp.dot(m, wrel_ref[r], preferred_element_type=jnp.float32)
  h_out[...] = jnp.maximum(acc, 0.0)


def _tc_layer(h, wroot, b, cnt4, agg, wrel):
  return pl.pallas_call(
      _tc_layer_body,
      grid=(GRID,),
      in_specs=[
          pl.BlockSpec((BN, D), lambda i: (i, 0)),
          _full((D, D)),
          _full((1, D)),
          pl.BlockSpec((NC, R, BN, 1), lambda i: (0, 0, i, 0)),
          pl.BlockSpec((NC, R, BN, D), lambda i: (0, 0, i, 0)),
          _full((R, D, D)),
      ],
      out_specs=pl.BlockSpec((BN, D), lambda i: (i, 0)),
      out_shape=jax.ShapeDtypeStruct((NPAD, D), jnp.float32),
  )(h, wroot, b, cnt4, agg, wrel)


def _prep_edges(ei):
  pad = EPAD - E
  fill = jnp.arange(pad, dtype=jnp.int32)
  src = jnp.concatenate([ei[0], fill % N])
  dst = jnp.concatenate([ei[1], N + fill % (NPAD - N)])
  return (src.reshape(NW, NCH, CH), dst.reshape(NW, NCH, CH))


@jax.jit
def kernel(x_node, edge_index_r0, edge_index_r1, edge_index_r2,
           Wrel0, Wroot0, b0, Wrel1, Wroot1, b1, Wrel2, Wroot2, b2):
  s0, d0 = _prep_edges(edge_index_r0)
  s1, d1 = _prep_edges(edge_index_r1)
  s2, d2 = _prep_edges(edge_index_r2)

  x = jnp.concatenate(
      [x_node, jnp.zeros((NPAD - N, D), jnp.float32)], axis=0)
  b0r = b0.reshape(1, D)
  b1r = b1.reshape(1, D)
  b2r = b2.reshape(1, D)

  agg0, cnt = _sc_edge_cnt(x, s0, s1, s2, d0, d1, d2)
  cnt4 = cnt.reshape(NC, R, NPAD, 1)
  h1 = _tc_layer(x, Wroot0, b0r, cnt4, agg0, Wrel0)

  agg1 = _sc_edge(h1, s0, s1, s2, d0, d1, d2)
  h2 = _tc_layer(h1, Wroot1, b1r, cnt4, agg1, Wrel1)

  agg2 = _sc_edge(h2, s0, s1, s2, d0, d1, d2)
  h3 = _tc_layer(h2, Wroot2, b2r, cnt4, agg2, Wrel2)
  return h3[:N]


# speedup vs baseline: 1.0051x; 1.0051x over previous
import functools

import jax
import jax.numpy as jnp
from jax import lax
from jax.experimental import pallas as pl
from jax.experimental.pallas import tpu as pltpu
from jax.experimental.pallas import tpu_sc as plsc

N = 10000
E = 106666
D = 128
R = 3

NC = 2
NS = 16
NW = NC * NS

CH = 128
NCH = 27
EPT = NCH * CH
EPAD = EPT * NW

NPAD = 10240
RPT = NPAD // NS
NBUF = 2
DS = 0
ZR = 16

BN = 1024
GRID = NPAD // BN


def _make_sc_edge(compute_cnt: bool):
  mesh = plsc.VectorSubcoreMesh(core_axis_name="c", subcore_axis_name="s",
                                num_cores=NC, num_subcores=NS)

  out_type = [jax.ShapeDtypeStruct((NC, R, NPAD, D), jnp.float32)]
  if compute_cnt:
    out_type.append(jax.ShapeDtypeStruct((NC * R * NPAD,), jnp.float32))

  scratch = (
      [pltpu.VMEM((NCH, CH), jnp.int32)] * 2 +
      [pltpu.VMEM((CH, D), jnp.float32)] * NBUF +
      [pltpu.VMEM((ZR, D), jnp.float32)] +
      [pltpu.VMEM_SHARED((NPAD, D), jnp.float32)] +
      [pltpu.SemaphoreType.DMA] * (2 * NBUF + 2)
  )
  if compute_cnt:
    scratch += [
        pltpu.VMEM((CH,), jnp.float32),
        pltpu.VMEM((RPT,), jnp.float32),
        pltpu.SemaphoreType.DMA,
        pltpu.VMEM_SHARED((NPAD,), jnp.float32),
        pltpu.VMEM_SHARED((NPAD,), jnp.float32),
        pltpu.VMEM_SHARED((NPAD,), jnp.float32),
    ]

  def body(*refs):
    (htab, s0, s1, s2, d0, d1, d2) = refs[:7]
    pos = 7
    agg_out = refs[pos]; pos += 1
    if compute_cnt:
      cnt_out = refs[pos]; pos += 1
    src_v, dst_v = refs[pos:pos + 2]
    pos += 2
    rows = refs[pos:pos + NBUF]
    pos += NBUF
    zeros_v, agg_sh = refs[pos:pos + 2]
    pos += 2
    gsem = refs[pos:pos + NBUF]
    pos += NBUF
    ssem = refs[pos:pos + NBUF]
    pos += NBUF
    zsem, xsem = refs[pos:pos + 2]
    pos += 2
    if compute_cnt:
      ones_v, cstage_v, csem, c0_sh, c1_sh, c2_sh = refs[pos:pos + 6]
      cnt_shs = (c0_sh, c1_sh, c2_sh)

    c = lax.axis_index("c")
    tid = lax.axis_index("s")
    wid = tid * NC + c

    zv = jnp.zeros((16,), jnp.float32)

    def zrow(i, carry):
      for j in range(D // 16):
        zeros_v[i, pl.ds(j * 16, 16)] = zv
      return carry

    lax.fori_loop(0, ZR, zrow, 0)

    if compute_cnt:
      ov = jnp.full((16,), 1.0, jnp.float32)
      for j in range(CH // 16):
        ones_v[pl.ds(j * 16, 16)] = ov

    ss = (s0, s1, s2)
    ds = (d0, d1, d2)

    pltpu.sync_copy(ss[0].at[wid], src_v)
    pltpu.sync_copy(ds[0].at[wid], dst_v)

    for r in range(R):
      if r == 0:
        zd = [
            pltpu.async_copy(
                zeros_v, agg_sh.at[pl.ds(tid * RPT + k * ZR, ZR)], zsem)
            for k in range(RPT // ZR)
        ]
        for d in zd:
          d.wait()
      if compute_cnt:
        czd = [
            pltpu.async_copy(
                zeros_v.at[k % ZR],
                cnt_shs[r].at[pl.ds(tid * RPT + k * D, D)], zsem)
            for k in range(RPT // D)
        ]
        for d in czd:
          d.wait()

      gd = [None] * NCH
      for j in range(min(NBUF - DS, NCH)):
        gd[j] = pltpu.async_copy(
            htab.at[src_v.at[j]], rows[j % NBUF], gsem[j % NBUF])
      plsc.subcore_barrier()

      cds = []
      for j in range(NCH):
        b = j % NBUF
        gd[j].wait()
        if compute_cnt:
          cds.append(pltpu.async_copy(
              ones_v, cnt_shs[r].at[dst_v.at[j]], csem, add=True))
        pltpu.async_copy(
            rows[b], agg_sh.at[dst_v.at[j]], ssem[b], add=True).wait()
        if j + NBUF < NCH:
          gd[j + NBUF] = pltpu.async_copy(
              htab.at[src_v.at[j + NBUF]], rows[b], gsem[b])
      for d in cds:
        d.wait()
      plsc.subcore_barrier()

      ipf = []
      if r + 1 < R:
        ipf.append(pltpu.async_copy(ss[r + 1].at[wid], src_v, xsem))
        ipf.append(pltpu.async_copy(ds[r + 1].at[wid], dst_v, xsem))
      nd = RPT // CH
      dd = [None] * nd
      od = [None] * nd
      zd2 = []
      dd[0] = pltpu.async_copy(
          agg_sh.at[pl.ds(tid * RPT, CH)], rows[0], gsem[0])
      for k in range(nd):
        b = k % NBUF
        dd[k].wait()
        if r + 1 < R:
          for z in range(CH // ZR):
            zd2.append(pltpu.async_copy(
                zeros_v,
                agg_sh.at[pl.ds(tid * RPT + k * CH + z * ZR, ZR)], zsem))
        od[k] = pltpu.async_copy(
            rows[b], agg_out.at[c, r, pl.ds(tid * RPT + k * CH, CH)], ssem[b])
        if k + 1 < nd:
          if k - 1 >= 0:
            od[k - 1].wait()
          nb = (k + 1) % NBUF
          dd[k + 1] = pltpu.async_copy(
              agg_sh.at[pl.ds(tid * RPT + (k + 1) * CH, CH)], rows[nb],
              gsem[nb])
      for k in range(max(0, nd - 2), nd):
        od[k].wait()
      for d in zd2:
        d.wait()
      for d in ipf:
        d.wait()
      if compute_cnt:
        pltpu.sync_copy(cnt_shs[r].at[pl.ds(tid * RPT, RPT)], cstage_v)
        base = (c * R + r) * NPAD + tid * RPT
        pltpu.sync_copy(cstage_v, cnt_out.at[pl.ds(base, RPT)])

  out = tuple(out_type) if compute_cnt else out_type[0]
  return pl.kernel(body, out_type=out, mesh=mesh, scratch_types=scratch)


_sc_edge_cnt = _make_sc_edge(True)
_sc_edge = _make_sc_edge(False)


def _full(shape):
  return pl.BlockSpec(shape, lambda i: tuple(0 for _ in shape))


def _tc_layer_body(h_ref, wroot_ref, b_ref, cnt_ref, agg_ref, wrel_ref,
                   h_out):
  h = h_ref[...]
  acc = jnp.dot(h, wroot_ref[...], preferred_element_type=jnp.float32)
  acc = acc + b_ref[...]
  for r in range(R):
    a = agg_ref[0, r] + agg_ref[1, r]
    cc = cnt_ref[0, r] + cnt_ref[1, r]
    inv = 1.0 / jnp.maximum(cc, 1.0)
    m = a * inv
    acc = acc + jnp.dot(m, wrel_ref[r], preferred_element_type=jnp.float32)
  h_out[...] = jnp.maximum(acc, 0.0)


def _tc_layer(h, wroot, b, cnt4, agg, wrel):
  return pl.pallas_call(
      _tc_layer_body,
      grid=(GRID,),
      in_specs=[
          pl.BlockSpec((BN, D), lambda i: (i, 0)),
          _full((D, D)),
          _full((1, D)),
          pl.BlockSpec((NC, R, BN, 1), lambda i: (0, 0, i, 0)),
          pl.BlockSpec((NC, R, BN, D), lambda i: (0, 0, i, 0)),
          _full((R, D, D)),
      ],
      out_specs=pl.BlockSpec((BN, D), lambda i: (i, 0)),
      out_shape=jax.ShapeDtypeStruct((NPAD, D), jnp.float32),
  )(h, wroot, b, cnt4, agg, wrel)


def _prep_edges(ei):
  pad = EPAD - E
  fill = jnp.arange(pad, dtype=jnp.int32)
  src = jnp.concatenate([ei[0], fill % N])
  dst = jnp.concatenate([ei[1], N + fill % (NPAD - N)])
  return (src.reshape(NW, NCH, CH), dst.reshape(NW, NCH, CH))


@jax.jit
def kernel(x_node, edge_index_r0, edge_index_r1, edge_index_r2,
           Wrel0, Wroot0, b0, Wrel1, Wroot1, b1, Wrel2, Wroot2, b2):
  s0, d0 = _prep_edges(edge_index_r0)
  s1, d1 = _prep_edges(edge_index_r1)
  s2, d2 = _prep_edges(edge_index_r2)

  x = jnp.concatenate(
      [x_node, jnp.zeros((NPAD - N, D), jnp.float32)], axis=0)
  b0r = b0.reshape(1, D)
  b1r = b1.reshape(1, D)
  b2r = b2.reshape(1, D)

  agg0, cnt = _sc_edge_cnt(x, s0, s1, s2, d0, d1, d2)
  cnt4 = cnt.reshape(NC, R, NPAD, 1)
  h1 = _tc_layer(x, Wroot0, b0r, cnt4, agg0, Wrel0)

  agg1 = _sc_edge(h1, s0, s1, s2, d0, d1, d2)
  h2 = _tc_layer(h1, Wroot1, b1r, cnt4, agg1, Wrel1)

  agg2 = _sc_edge(h2, s0, s1, s2, d0, d1, d2)
  h3 = _tc_layer(h2, Wroot2, b2r, cnt4, agg2, Wrel2)
  return h3[:N]


# speedup vs baseline: 1.0301x; 1.0248x over previous
import functools

import jax
import jax.numpy as jnp
from jax import lax
from jax.experimental import pallas as pl
from jax.experimental.pallas import tpu as pltpu
from jax.experimental.pallas import tpu_sc as plsc

N = 10000
E = 106666
D = 128
R = 3

NC = 2
NS = 16
NW = NC * NS

CH = 128
NCH = 27
EPT = NCH * CH
EPAD = EPT * NW

NPAD = 10240
RPT = NPAD // NS
NBUF = 2
DS = 0
ZR = 16

BN = 1000
GRID = N // BN


def _make_sc_edge(compute_cnt: bool):
  mesh = plsc.VectorSubcoreMesh(core_axis_name="c", subcore_axis_name="s",
                                num_cores=NC, num_subcores=NS)

  out_type = [jax.ShapeDtypeStruct((NC, R, NPAD, D), jnp.float32)]
  if compute_cnt:
    out_type.append(jax.ShapeDtypeStruct((NC * R * NPAD,), jnp.float32))

  scratch = (
      [pltpu.VMEM((NCH, CH), jnp.int32)] * 2 +
      [pltpu.VMEM((CH, D), jnp.float32)] * NBUF +
      [pltpu.VMEM((ZR, D), jnp.float32)] +
      [pltpu.VMEM_SHARED((NPAD, D), jnp.float32)] +
      [pltpu.SemaphoreType.DMA] * (2 * NBUF + 2)
  )
  if compute_cnt:
    scratch += [
        pltpu.VMEM((CH,), jnp.float32),
        pltpu.VMEM((RPT,), jnp.float32),
        pltpu.SemaphoreType.DMA,
        pltpu.VMEM_SHARED((NPAD,), jnp.float32),
        pltpu.VMEM_SHARED((NPAD,), jnp.float32),
        pltpu.VMEM_SHARED((NPAD,), jnp.float32),
    ]

  def body(*refs):
    (htab, s0, s1, s2, d0, d1, d2) = refs[:7]
    pos = 7
    agg_out = refs[pos]; pos += 1
    if compute_cnt:
      cnt_out = refs[pos]; pos += 1
    src_v, dst_v = refs[pos:pos + 2]
    pos += 2
    rows = refs[pos:pos + NBUF]
    pos += NBUF
    zeros_v, agg_sh = refs[pos:pos + 2]
    pos += 2
    gsem = refs[pos:pos + NBUF]
    pos += NBUF
    ssem = refs[pos:pos + NBUF]
    pos += NBUF
    zsem, xsem = refs[pos:pos + 2]
    pos += 2
    if compute_cnt:
      ones_v, cstage_v, csem, c0_sh, c1_sh, c2_sh = refs[pos:pos + 6]
      cnt_shs = (c0_sh, c1_sh, c2_sh)

    c = lax.axis_index("c")
    tid = lax.axis_index("s")
    wid = tid * NC + c

    zv = jnp.zeros((16,), jnp.float32)

    def zrow(i, carry):
      for j in range(D // 16):
        zeros_v[i, pl.ds(j * 16, 16)] = zv
      return carry

    lax.fori_loop(0, ZR, zrow, 0)

    if compute_cnt:
      ov = jnp.full((16,), 1.0, jnp.float32)
      for j in range(CH // 16):
        ones_v[pl.ds(j * 16, 16)] = ov

    ss = (s0, s1, s2)
    ds = (d0, d1, d2)

    pltpu.sync_copy(ss[0].at[wid], src_v)
    pltpu.sync_copy(ds[0].at[wid], dst_v)

    for r in range(R):
      if r == 0:
        zd = [
            pltpu.async_copy(
                zeros_v, agg_sh.at[pl.ds(tid * RPT + k * ZR, ZR)], zsem)
            for k in range(RPT // ZR)
        ]
        for d in zd:
          d.wait()
      if compute_cnt:
        czd = [
            pltpu.async_copy(
                zeros_v.at[k % ZR],
                cnt_shs[r].at[pl.ds(tid * RPT + k * D, D)], zsem)
            for k in range(RPT // D)
        ]
        for d in czd:
          d.wait()

      gd = [None] * NCH
      for j in range(min(NBUF - DS, NCH)):
        gd[j] = pltpu.async_copy(
            htab.at[src_v.at[j]], rows[j % NBUF], gsem[j % NBUF])
      plsc.subcore_barrier()

      cds = []
      for j in range(NCH):
        b = j % NBUF
        gd[j].wait()
        if compute_cnt:
          cds.append(pltpu.async_copy(
              ones_v, cnt_shs[r].at[dst_v.at[j]], csem, add=True))
        pltpu.async_copy(
            rows[b], agg_sh.at[dst_v.at[j]], ssem[b], add=True).wait()
        if j + NBUF < NCH:
          gd[j + NBUF] = pltpu.async_copy(
              htab.at[src_v.at[j + NBUF]], rows[b], gsem[b])
      for d in cds:
        d.wait()
      plsc.subcore_barrier()

      ipf = []
      if r + 1 < R:
        ipf.append(pltpu.async_copy(ss[r + 1].at[wid], src_v, xsem))
        ipf.append(pltpu.async_copy(ds[r + 1].at[wid], dst_v, xsem))
      nd = RPT // CH
      dd = [None] * nd
      od = [None] * nd
      zd2 = []
      dd[0] = pltpu.async_copy(
          agg_sh.at[pl.ds(tid * RPT, CH)], rows[0], gsem[0])
      for k in range(nd):
        b = k % NBUF
        dd[k].wait()
        if r + 1 < R:
          for z in range(CH // ZR):
            zd2.append(pltpu.async_copy(
                zeros_v,
                agg_sh.at[pl.ds(tid * RPT + k * CH + z * ZR, ZR)], zsem))
        od[k] = pltpu.async_copy(
            rows[b], agg_out.at[c, r, pl.ds(tid * RPT + k * CH, CH)], ssem[b])
        if k + 1 < nd:
          if k - 1 >= 0:
            od[k - 1].wait()
          nb = (k + 1) % NBUF
          dd[k + 1] = pltpu.async_copy(
              agg_sh.at[pl.ds(tid * RPT + (k + 1) * CH, CH)], rows[nb],
              gsem[nb])
      for k in range(max(0, nd - 2), nd):
        od[k].wait()
      for d in zd2:
        d.wait()
      for d in ipf:
        d.wait()
      if compute_cnt:
        pltpu.sync_copy(cnt_shs[r].at[pl.ds(tid * RPT, RPT)], cstage_v)
        base = (c * R + r) * NPAD + tid * RPT
        pltpu.sync_copy(cstage_v, cnt_out.at[pl.ds(base, RPT)])

  out = tuple(out_type) if compute_cnt else out_type[0]
  return pl.kernel(body, out_type=out, mesh=mesh, scratch_types=scratch)


_sc_edge_cnt = _make_sc_edge(True)
_sc_edge = _make_sc_edge(False)


def _full(shape):
  return pl.BlockSpec(shape, lambda i: tuple(0 for _ in shape))


def _tc_layer_body(h_ref, wroot_ref, b_ref, cnt_ref, agg_ref, wrel_ref,
                   h_out):
  h = h_ref[...]
  acc = jnp.dot(h, wroot_ref[...], preferred_element_type=jnp.float32)
  acc = acc + b_ref[...]
  for r in range(R):
    a = agg_ref[0, r] + agg_ref[1, r]
    cc = cnt_ref[0, r] + cnt_ref[1, r]
    inv = 1.0 / jnp.maximum(cc, 1.0)
    m = a * inv
    acc = acc + jnp.dot(m, wrel_ref[r], preferred_element_type=jnp.float32)
  h_out[...] = jnp.maximum(acc, 0.0)


def _tc_layer(h, wroot, b, cnt4, agg, wrel):
  return pl.pallas_call(
      _tc_layer_body,
      grid=(GRID,),
      in_specs=[
          pl.BlockSpec((BN, D), lambda i: (i, 0)),
          _full((D, D)),
          _full((1, D)),
          pl.BlockSpec((NC, R, BN, 1), lambda i: (0, 0, i, 0)),
          pl.BlockSpec((NC, R, BN, D), lambda i: (0, 0, i, 0)),
          _full((R, D, D)),
      ],
      out_specs=pl.BlockSpec((BN, D), lambda i: (i, 0)),
      out_shape=jax.ShapeDtypeStruct((N, D), jnp.float32),
  )(h, wroot, b, cnt4, agg, wrel)


def _prep_edges(ei):
  pad = EPAD - E
  fill = jnp.arange(pad, dtype=jnp.int32)
  src = jnp.concatenate([ei[0], fill % N])
  dst = jnp.concatenate([ei[1], N + fill % (NPAD - N)])
  return (src.reshape(NW, NCH, CH), dst.reshape(NW, NCH, CH))


@jax.jit
def kernel(x_node, edge_index_r0, edge_index_r1, edge_index_r2,
           Wrel0, Wroot0, b0, Wrel1, Wroot1, b1, Wrel2, Wroot2, b2):
  s0, d0 = _prep_edges(edge_index_r0)
  s1, d1 = _prep_edges(edge_index_r1)
  s2, d2 = _prep_edges(edge_index_r2)

  b0r = b0.reshape(1, D)
  b1r = b1.reshape(1, D)
  b2r = b2.reshape(1, D)

  agg0, cnt = _sc_edge_cnt(x_node, s0, s1, s2, d0, d1, d2)
  cnt4 = cnt.reshape(NC, R, NPAD, 1)
  h1 = _tc_layer(x_node, Wroot0, b0r, cnt4, agg0, Wrel0)

  agg1 = _sc_edge(h1, s0, s1, s2, d0, d1, d2)
  h2 = _tc_layer(h1, Wroot1, b1r, cnt4, agg1, Wrel1)

  agg2 = _sc_edge(h2, s0, s1, s2, d0, d1, d2)
  h3 = _tc_layer(h2, Wroot2, b2r, cnt4, agg2, Wrel2)
  return h3


# speedup vs baseline: 1.0310x; 1.0009x over previous
import jax
import jax.numpy as jnp
from jax import lax
from jax.experimental import pallas as pl
from jax.experimental.pallas import tpu as pltpu
from jax.experimental.pallas import tpu_sc as plsc

N = 10000
E = 106666
D = 128
R = 3

NC = 2
NS = 16
NW = NC * NS

CH = 128
NCH = 27
EPT = NCH * CH
EPAD = EPT * NW

NPAD = 10240
RPT = NPAD // NS
NBUF = 2
DS = 0
ZR = 16

BN = 2000
GRID = N // BN


def _make_sc_edge(compute_cnt: bool):
  mesh = plsc.VectorSubcoreMesh(core_axis_name="c", subcore_axis_name="s",
                                num_cores=NC, num_subcores=NS)

  out_type = [jax.ShapeDtypeStruct((NC, R, NPAD, D), jnp.float32)]
  if compute_cnt:
    out_type.append(jax.ShapeDtypeStruct((NC * R * NPAD,), jnp.float32))

  scratch = (
      [pltpu.VMEM((NCH, CH), jnp.int32)] * 2 +
      [pltpu.VMEM((CH, D), jnp.float32)] * NBUF +
      [pltpu.VMEM((ZR, D), jnp.float32)] +
      [pltpu.VMEM_SHARED((NPAD, D), jnp.float32)] +
      [pltpu.SemaphoreType.DMA] * (2 * NBUF + 2)
  )
  if compute_cnt:
    scratch += [
        pltpu.VMEM((CH,), jnp.float32),
        pltpu.VMEM((RPT,), jnp.float32),
        pltpu.SemaphoreType.DMA,
        pltpu.VMEM_SHARED((NPAD,), jnp.float32),
        pltpu.VMEM_SHARED((NPAD,), jnp.float32),
        pltpu.VMEM_SHARED((NPAD,), jnp.float32),
    ]

  def body(*refs):
    (htab, s0, s1, s2, d0, d1, d2) = refs[:7]
    pos = 7
    agg_out = refs[pos]; pos += 1
    if compute_cnt:
      cnt_out = refs[pos]; pos += 1
    src_v, dst_v = refs[pos:pos + 2]
    pos += 2
    rows = refs[pos:pos + NBUF]
    pos += NBUF
    zeros_v, agg_sh = refs[pos:pos + 2]
    pos += 2
    gsem = refs[pos:pos + NBUF]
    pos += NBUF
    ssem = refs[pos:pos + NBUF]
    pos += NBUF
    zsem, xsem = refs[pos:pos + 2]
    pos += 2
    if compute_cnt:
      ones_v, cstage_v, csem, c0_sh, c1_sh, c2_sh = refs[pos:pos + 6]
      cnt_shs = (c0_sh, c1_sh, c2_sh)

    c = lax.axis_index("c")
    tid = lax.axis_index("s")
    wid = tid * NC + c

    zv = jnp.zeros((16,), jnp.float32)

    def zrow(i, carry):
      for j in range(D // 16):
        zeros_v[i, pl.ds(j * 16, 16)] = zv
      return carry

    lax.fori_loop(0, ZR, zrow, 0)

    if compute_cnt:
      ov = jnp.full((16,), 1.0, jnp.float32)
      for j in range(CH // 16):
        ones_v[pl.ds(j * 16, 16)] = ov

    ss = (s0, s1, s2)
    ds = (d0, d1, d2)

    pltpu.sync_copy(ss[0].at[wid], src_v)
    pltpu.sync_copy(ds[0].at[wid], dst_v)

    for r in range(R):
      if r == 0:
        zd = [
            pltpu.async_copy(
                zeros_v, agg_sh.at[pl.ds(tid * RPT + k * ZR, ZR)], zsem)
            for k in range(RPT // ZR)
        ]
        for d in zd:
          d.wait()
      if compute_cnt:
        czd = [
            pltpu.async_copy(
                zeros_v.at[k % ZR],
                cnt_shs[r].at[pl.ds(tid * RPT + k * D, D)], zsem)
            for k in range(RPT // D)
        ]
        for d in czd:
          d.wait()

      gd = [None] * NCH
      for j in range(min(NBUF - DS, NCH)):
        gd[j] = pltpu.async_copy(
            htab.at[src_v.at[j]], rows[j % NBUF], gsem[j % NBUF])
      plsc.subcore_barrier()

      cds = []
      for j in range(NCH):
        b = j % NBUF
        gd[j].wait()
        if compute_cnt:
          cds.append(pltpu.async_copy(
              ones_v, cnt_shs[r].at[dst_v.at[j]], csem, add=True))
        pltpu.async_copy(
            rows[b], agg_sh.at[dst_v.at[j]], ssem[b], add=True).wait()
        if j + NBUF < NCH:
          gd[j + NBUF] = pltpu.async_copy(
              htab.at[src_v.at[j + NBUF]], rows[b], gsem[b])
      for d in cds:
        d.wait()
      plsc.subcore_barrier()

      ipf = []
      if r + 1 < R:
        ipf.append(pltpu.async_copy(ss[r + 1].at[wid], src_v, xsem))
        ipf.append(pltpu.async_copy(ds[r + 1].at[wid], dst_v, xsem))
      nd = RPT // CH
      dd = [None] * nd
      od = [None] * nd
      zd2 = []
      dd[0] = pltpu.async_copy(
          agg_sh.at[pl.ds(tid * RPT, CH)], rows[0], gsem[0])
      for k in range(nd):
        b = k % NBUF
        dd[k].wait()
        if r + 1 < R:
          for z in range(CH // ZR):
            zd2.append(pltpu.async_copy(
                zeros_v,
                agg_sh.at[pl.ds(tid * RPT + k * CH + z * ZR, ZR)], zsem))
        od[k] = pltpu.async_copy(
            rows[b], agg_out.at[c, r, pl.ds(tid * RPT + k * CH, CH)], ssem[b])
        if k + 1 < nd:
          if k - 1 >= 0:
            od[k - 1].wait()
          nb = (k + 1) % NBUF
          dd[k + 1] = pltpu.async_copy(
              agg_sh.at[pl.ds(tid * RPT + (k + 1) * CH, CH)], rows[nb],
              gsem[nb])
      for k in range(max(0, nd - 2), nd):
        od[k].wait()
      for d in zd2:
        d.wait()
      for d in ipf:
        d.wait()
      if compute_cnt:
        pltpu.sync_copy(cnt_shs[r].at[pl.ds(tid * RPT, RPT)], cstage_v)
        base = (c * R + r) * NPAD + tid * RPT
        pltpu.sync_copy(cstage_v, cnt_out.at[pl.ds(base, RPT)])

  out = tuple(out_type) if compute_cnt else out_type[0]
  return pl.kernel(body, out_type=out, mesh=mesh, scratch_types=scratch)


_sc_edge_cnt = _make_sc_edge(True)
_sc_edge = _make_sc_edge(False)


def _full(shape):
  return pl.BlockSpec(shape, lambda i: tuple(0 for _ in shape))


def _tc_layer_body(h_ref, wroot_ref, b_ref, cnt_ref, agg_ref, wrel_ref,
                   h_out):
  h = h_ref[...]
  acc = jnp.dot(h, wroot_ref[...], preferred_element_type=jnp.float32)
  acc = acc + b_ref[...]
  for r in range(R):
    a = agg_ref[0, r] + agg_ref[1, r]
    cc = cnt_ref[0, r] + cnt_ref[1, r]
    inv = 1.0 / jnp.maximum(cc, 1.0)
    m = a * inv
    acc = acc + jnp.dot(m, wrel_ref[r], preferred_element_type=jnp.float32)
  h_out[...] = jnp.maximum(acc, 0.0)


def _tc_layer(h, wroot, b, cnt4, agg, wrel):
  return pl.pallas_call(
      _tc_layer_body,
      grid=(GRID,),
      in_specs=[
          pl.BlockSpec((BN, D), lambda i: (i, 0)),
          _full((D, D)),
          _full((1, D)),
          pl.BlockSpec((NC, R, BN, 1), lambda i: (0, 0, i, 0)),
          pl.BlockSpec((NC, R, BN, D), lambda i: (0, 0, i, 0)),
          _full((R, D, D)),
      ],
      out_specs=pl.BlockSpec((BN, D), lambda i: (i, 0)),
      out_shape=jax.ShapeDtypeStruct((N, D), jnp.float32),
  )(h, wroot, b, cnt4, agg, wrel)


def _prep_edges(ei):
  pad = EPAD - E
  fill = jnp.arange(pad, dtype=jnp.int32)
  src = jnp.concatenate([ei[0], fill % N])
  dst = jnp.concatenate([ei[1], N + fill % (NPAD - N)])
  return (src.reshape(NW, NCH, CH), dst.reshape(NW, NCH, CH))


@jax.jit
def kernel(x_node, edge_index_r0, edge_index_r1, edge_index_r2,
           Wrel0, Wroot0, b0, Wrel1, Wroot1, b1, Wrel2, Wroot2, b2):
  s0, d0 = _prep_edges(edge_index_r0)
  s1, d1 = _prep_edges(edge_index_r1)
  s2, d2 = _prep_edges(edge_index_r2)

  b0r = b0.reshape(1, D)
  b1r = b1.reshape(1, D)
  b2r = b2.reshape(1, D)

  agg0, cnt = _sc_edge_cnt(x_node, s0, s1, s2, d0, d1, d2)
  cnt4 = cnt.reshape(NC, R, NPAD, 1)
  h1 = _tc_layer(x_node, Wroot0, b0r, cnt4, agg0, Wrel0)

  agg1 = _sc_edge(h1, s0, s1, s2, d0, d1, d2)
  h2 = _tc_layer(h1, Wroot1, b1r, cnt4, agg1, Wrel1)

  agg2 = _sc_edge(h2, s0, s1, s2, d0, d1, d2)
  h3 = _tc_layer(h2, Wroot2, b2r, cnt4, agg2, Wrel2)
  return h3
